# baseline (speedup 1.0000x reference)
import functools

import jax
import jax.numpy as jnp
from jax.experimental import pallas as pl
from jax.experimental.pallas import tpu as pltpu

_N_EMB = 8192
_DIM = 256
_BETA = 0.25
_EPS = 1e-05
_DIVERSITY = 0.8
_TM = 256

_DN0 = (((0,), (0,)), ((), ()))


def _vq_body(n_steps, m_total, x_ref, e_ref,
             out_ref, loss_ref, perp_ref,
             ebf_ref, etbf_ref, esq_ref, dwt_ref, sqerr_ref):
    i = pl.program_id(0)

    @pl.when(i == 0)
    def _prep():
        e = e_ref[...]
        et = jnp.transpose(e)
        ebf_ref[...] = e.astype(jnp.bfloat16)
        etbf_ref[...] = (et + et).astype(jnp.bfloat16)
        esq_ref[...] = jnp.sum(et * et, axis=0, keepdims=True)

    x = x_ref[...]
    xbf = x.astype(jnp.bfloat16)
    esq = esq_ref[...]
    mm = jnp.dot(xbf, etbf_ref[...],
                 preferred_element_type=jnp.float32)
    mx = jnp.max(mm - esq, axis=1, keepdims=True)
    p = jnp.exp(mm - esq - mx)
    s = jnp.sum(p, axis=1, keepdims=True)
    encbf = (p * (1.0 / s)).astype(jnp.bfloat16)

    q = jnp.dot(encbf, ebf_ref[...],
                preferred_element_type=jnp.float32)
    d = q - x
    out_ref[...] = x + d

    sq_prev = jnp.where(i == 0, 0.0, sqerr_ref[...])
    sqerr = sq_prev + jnp.sum(d * d, axis=(0, 1), keepdims=True)
    sqerr_ref[...] = sqerr
    xaug = jnp.concatenate(
        [xbf, jnp.ones((_TM, 8), jnp.bfloat16)], axis=1)
    dwt_prev = jnp.where(i == 0, 0.0, dwt_ref[...])
    dwt = dwt_prev + jax.lax.dot_general(
        xaug, encbf, _DN0, preferred_element_type=jnp.float32)
    dwt_ref[...] = dwt

    @pl.when(i == n_steps - 1)
    def _scalars():
        colsum = dwt[_DIM:_DIM + 1, :]
        avg_probs = colsum / m_total
        entropy = -jnp.sum(avg_probs * jnp.log(avg_probs + 1e-10),
                           axis=(0, 1), keepdims=True)
        usage = 0.01 * colsum
        up = usage / (jnp.sum(usage, axis=(0, 1), keepdims=True) + 1e-5)
        diversity = -jnp.sum(up * jnp.log(up + 1e-10),
                             axis=(0, 1), keepdims=True)
        ema_cs = 0.01 * colsum
        n = jnp.sum(ema_cs, axis=(0, 1), keepdims=True)
        cs = (ema_cs + _EPS) / (n + _N_EMB * _EPS) * n
        ema_wt = 0.01 * dwt[0:_DIM, :]
        r = jnp.sum(ema_wt * ema_wt, axis=0, keepdims=True)
        reg = jnp.sum(r / (cs * cs), axis=(0, 1), keepdims=True)
        mse = sqerr / (m_total * _DIM)
        loss_ref[...] = (mse + _BETA * mse + reg
                         + _DIVERSITY * (entropy + diversity))
        perp_ref[...] = jnp.exp(entropy)


def kernel(x, embeddings):
    xf = x.reshape(-1, _DIM)
    m = xf.shape[0]
    n_steps = m // _TM

    out, loss, perp = pl.pallas_call(
        functools.partial(_vq_body, n_steps, float(m)),
        grid=(n_steps,),
        in_specs=[
            pl.BlockSpec((_TM, _DIM), lambda i: (i, 0)),
            pl.BlockSpec((_N_EMB, _DIM), lambda i: (0, 0)),
        ],
        out_specs=[
            pl.BlockSpec((_TM, _DIM), lambda i: (i, 0)),
            pl.BlockSpec((1, 1), lambda i: (0, 0)),
            pl.BlockSpec((1, 1), lambda i: (0, 0)),
        ],
        out_shape=[
            jax.ShapeDtypeStruct((m, _DIM), jnp.float32),
            jax.ShapeDtypeStruct((1, 1), jnp.float32),
            jax.ShapeDtypeStruct((1, 1), jnp.float32),
        ],
        scratch_shapes=[
            pltpu.VMEM((_N_EMB, _DIM), jnp.bfloat16),
            pltpu.VMEM((_DIM, _N_EMB), jnp.bfloat16),
            pltpu.VMEM((1, _N_EMB), jnp.float32),
            pltpu.VMEM((_DIM + 8, _N_EMB), jnp.float32),
            pltpu.VMEM((1, 1), jnp.float32),
        ],
        compiler_params=pltpu.CompilerParams(
            dimension_semantics=("arbitrary",),
        ),
    )(xf, embeddings)
    return out.reshape(x.shape), loss[0, 0], perp[0, 0]


# speedup vs baseline: 1.0688x; 1.0688x over previous
import functools

import jax
import jax.numpy as jnp
from jax.experimental import pallas as pl
from jax.experimental.pallas import tpu as pltpu

_N_EMB = 8192
_DIM = 256
_BETA = 0.25
_EPS = 1e-05
_DIVERSITY = 0.8
_TM = 512

_DN0 = (((0,), (0,)), ((), ()))


def _vq_body(n_steps, m_total, x_ref, e_ref,
             out_ref, loss_ref, perp_ref,
             ebf_ref, etbf_ref, esq_ref, dwt_ref, sqerr_ref):
    i = pl.program_id(0)

    @pl.when(i == 0)
    def _prep():
        e = e_ref[...]
        et = jnp.transpose(e)
        ebf_ref[...] = e.astype(jnp.bfloat16)
        etbf_ref[...] = (et + et).astype(jnp.bfloat16)
        esq_ref[...] = jnp.sum(et * et, axis=0, keepdims=True)

    x = x_ref[...]
    xbf = x.astype(jnp.bfloat16)
    esq = esq_ref[...]
    mm = jnp.dot(xbf, etbf_ref[...],
                 preferred_element_type=jnp.float32)
    mx = jnp.max(mm - esq, axis=1, keepdims=True)
    p = jnp.exp(mm - esq - mx)
    s = jnp.sum(p, axis=1, keepdims=True)
    encbf = (p * (1.0 / s)).astype(jnp.bfloat16)

    q = jnp.dot(encbf, ebf_ref[...],
                preferred_element_type=jnp.float32)
    d = q - x
    out_ref[...] = x + d

    sq_prev = jnp.where(i == 0, 0.0, sqerr_ref[...])
    sqerr = sq_prev + jnp.sum(d * d, axis=(0, 1), keepdims=True)
    sqerr_ref[...] = sqerr
    xaug = jnp.concatenate(
        [xbf, jnp.ones((_TM, 8), jnp.bfloat16)], axis=1)
    dwt_prev = jnp.where(i == 0, 0.0, dwt_ref[...])
    dwt = dwt_prev + jax.lax.dot_general(
        xaug, encbf, _DN0, preferred_element_type=jnp.float32)
    dwt_ref[...] = dwt

    @pl.when(i == n_steps - 1)
    def _scalars():
        colsum = dwt[_DIM:_DIM + 1, :]
        avg_probs = colsum / m_total
        entropy = -jnp.sum(avg_probs * jnp.log(avg_probs + 1e-10),
                           axis=(0, 1), keepdims=True)
        usage = 0.01 * colsum
        up = usage / (jnp.sum(usage, axis=(0, 1), keepdims=True) + 1e-5)
        diversity = -jnp.sum(up * jnp.log(up + 1e-10),
                             axis=(0, 1), keepdims=True)
        ema_cs = 0.01 * colsum
        n = jnp.sum(ema_cs, axis=(0, 1), keepdims=True)
        cs = (ema_cs + _EPS) / (n + _N_EMB * _EPS) * n
        ema_wt = 0.01 * dwt[0:_DIM, :]
        r = jnp.sum(ema_wt * ema_wt, axis=0, keepdims=True)
        reg = jnp.sum(r / (cs * cs), axis=(0, 1), keepdims=True)
        mse = sqerr / (m_total * _DIM)
        loss_ref[...] = (mse + _BETA * mse + reg
                         + _DIVERSITY * (entropy + diversity))
        perp_ref[...] = jnp.exp(entropy)


def kernel(x, embeddings):
    xf = x.reshape(-1, _DIM)
    m = xf.shape[0]
    n_steps = m // _TM

    out, loss, perp = pl.pallas_call(
        functools.partial(_vq_body, n_steps, float(m)),
        grid=(n_steps,),
        in_specs=[
            pl.BlockSpec((_TM, _DIM), lambda i: (i, 0)),
            pl.BlockSpec((_N_EMB, _DIM), lambda i: (0, 0)),
        ],
        out_specs=[
            pl.BlockSpec((_TM, _DIM), lambda i: (i, 0)),
            pl.BlockSpec((1, 1), lambda i: (0, 0)),
            pl.BlockSpec((1, 1), lambda i: (0, 0)),
        ],
        out_shape=[
            jax.ShapeDtypeStruct((m, _DIM), jnp.float32),
            jax.ShapeDtypeStruct((1, 1), jnp.float32),
            jax.ShapeDtypeStruct((1, 1), jnp.float32),
        ],
        scratch_shapes=[
            pltpu.VMEM((_N_EMB, _DIM), jnp.bfloat16),
            pltpu.VMEM((_DIM, _N_EMB), jnp.bfloat16),
            pltpu.VMEM((1, _N_EMB), jnp.float32),
            pltpu.VMEM((_DIM + 8, _N_EMB), jnp.float32),
            pltpu.VMEM((1, 1), jnp.float32),
        ],
        compiler_params=pltpu.CompilerParams(
            dimension_semantics=("arbitrary",),
        ),
    )(xf, embeddings)
    return out.reshape(x.shape), loss[0, 0], perp[0, 0]
